# Initial kernel scaffold; baseline (speedup 1.0000x reference)
import jax
import jax.numpy as jnp
from jax.experimental import pallas as pl


def kernel(x, edge_index, relations, W1l, W1r, b1, g1, be1, W2l, W2r, b2, g2, be2, W3l, W3r, b3):
    raise NotImplementedError("write your pallas kernel here")


# speedup vs baseline: 3.1692x; 3.1692x over previous
import jax
import jax.numpy as jnp
from jax import lax
from jax.experimental import pallas as pl
from jax.experimental.pallas import tpu as pltpu
from jax.experimental.pallas import tpu_sc as plsc

N = 10000
E = 320000
D = 128

NC = 2
NS = 16
NW = NC * NS

C = 128
K = 80
E_PAD = NW * K * C
ROWS_PER_SUB = 632
NPAD = NS * ROWS_PER_SUB

import functools


def _mesh():
  return plsc.VectorSubcoreMesh(core_axis_name="c", subcore_axis_name="s")


def _seg_sum_body(h_hbm, src_hbm, dst_hbm, z_hbm, agg_out,
                  src_v, dst_v, msg_a, msg_b, agg_sh, sem_a, sem_b):
  c = lax.axis_index("c")
  s = lax.axis_index("s")
  wid = c * NS + s
  r0 = pl.multiple_of(s * ROWS_PER_SUB, 8)

  pltpu.sync_copy(src_hbm.at[wid], src_v)
  pltpu.sync_copy(dst_hbm.at[wid], dst_v)
  pltpu.sync_copy(z_hbm.at[pl.ds(r0, ROWS_PER_SUB)],
                  agg_sh.at[pl.ds(r0, ROWS_PER_SUB)])

  plsc.subcore_barrier()

  def gather(j, buf, sem):
    pltpu.async_copy(h_hbm.at[src_v.at[j]], buf, sem)

  def wait_gather(j, buf, sem):
    pltpu.make_async_copy(h_hbm.at[src_v.at[j]], buf, sem).wait()

  def chunk(j, _):
    gather(j, msg_a, sem_a)
    wait_gather(j, msg_a, sem_a)
    pltpu.sync_copy(msg_a, agg_sh.at[dst_v.at[j]], add=True)
    return 0

  lax.fori_loop(0, K, chunk, 0)

  plsc.subcore_barrier()

  pltpu.sync_copy(agg_sh.at[pl.ds(r0, ROWS_PER_SUB)],
                  agg_out.at[c, pl.ds(r0, ROWS_PER_SUB)])


@functools.cache
def _seg_sum():
  return pl.kernel(
      _seg_sum_body,
      out_type=jax.ShapeDtypeStruct((NC, NPAD, D), jnp.float32),
      mesh=_mesh(),
      scratch_types=[
          pltpu.VMEM((K, C), jnp.int32),
          pltpu.VMEM((K, C), jnp.int32),
          pltpu.VMEM((C, D), jnp.float32),
          pltpu.VMEM((C, D), jnp.float32),
          pltpu.VMEM_SHARED((NPAD, D), jnp.float32),
          pltpu.SemaphoreType.DMA,
          pltpu.SemaphoreType.DMA,
      ],
      name="sage_seg_sum")


def _deg_body(dst_hbm, z_hbm, ones_hbm, deg_out, dst_v, ones_v, deg_sh):
  c = lax.axis_index("c")
  s = lax.axis_index("s")
  wid = c * NS + s
  r0 = pl.multiple_of(s * ROWS_PER_SUB, 8)

  pltpu.sync_copy(z_hbm.at[pl.ds(r0, ROWS_PER_SUB)],
                  deg_sh.at[pl.ds(r0, ROWS_PER_SUB)])
  pltpu.sync_copy(ones_hbm, ones_v)
  pltpu.sync_copy(dst_hbm.at[wid], dst_v)

  plsc.subcore_barrier()

  def chunk(j, _):
    pltpu.sync_copy(ones_v, deg_sh.at[dst_v.at[j]], add=True)
    return 0

  lax.fori_loop(0, K, chunk, 0)

  plsc.subcore_barrier()

  pltpu.sync_copy(deg_sh.at[pl.ds(r0, ROWS_PER_SUB)],
                  deg_out.at[c, pl.ds(r0, ROWS_PER_SUB)])


@functools.cache
def _deg():
  return pl.kernel(
      _deg_body,
      out_type=jax.ShapeDtypeStruct((NC, NPAD, D), jnp.float32),
      mesh=_mesh(),
      scratch_types=[
          pltpu.VMEM((K, C), jnp.int32),
          pltpu.VMEM((C, D), jnp.float32),
          pltpu.VMEM_SHARED((NPAD, D), jnp.float32),
      ],
      name="sage_deg")


def _dense_body(agg_ref, scale_ref, h_ref, wl_ref, wr_ref, b_ref, g_ref,
                be_ref, o_ref):
  agg = agg_ref[0, :N, :] + agg_ref[1, :N, :]
  mean = agg * scale_ref[...]
  out = (jnp.dot(mean, wl_ref[...], preferred_element_type=jnp.float32)
         + jnp.dot(h_ref[...], wr_ref[...], preferred_element_type=jnp.float32)
         + b_ref[...])
  mu = jnp.mean(out, axis=0)
  var = jnp.mean((out - mu) ** 2, axis=0)
  inv = lax.rsqrt(var + 1e-5) * g_ref[...]
  o_ref[...] = jnp.maximum((out - mu) * inv + be_ref[...], 0.0)


def _scale_body(deg_ref, scale_ref):
  deg = deg_ref[0, :N, 0:1] + deg_ref[1, :N, 0:1]
  scale_ref[...] = 1.0 / jnp.maximum(deg, 1.0)


def _final_body(agg_ref, scale_ref, h_ref, wl_ref, wr_ref, b_ref, o_ref):
  agg = agg_ref[0, :N, :] + agg_ref[1, :N, :]
  mean = agg * scale_ref[...]
  out = (jnp.dot(mean, wl_ref[...], preferred_element_type=jnp.float32)
         + jnp.dot(h_ref[...], wr_ref[...], preferred_element_type=jnp.float32)
         + b_ref[...])
  m = jnp.max(out, axis=-1, keepdims=True)
  lse = jnp.log(jnp.sum(jnp.exp(out - m), axis=-1, keepdims=True))
  o_ref[...] = out - m - lse


_dense = pl.pallas_call(
    _dense_body, out_shape=jax.ShapeDtypeStruct((N, D), jnp.float32))
_scale = pl.pallas_call(
    _scale_body, out_shape=jax.ShapeDtypeStruct((N, 1), jnp.float32))
_final = pl.pallas_call(
    _final_body, out_shape=jax.ShapeDtypeStruct((N, 64), jnp.float32))


@jax.jit
def kernel(x, edge_index, relations, W1l, W1r, b1, g1, be1, W2l, W2r, b2, g2,
           be2, W3l, W3r, b3):
  del relations
  pad = E_PAD - E
  src = jnp.concatenate(
      [edge_index[0], jnp.zeros((pad,), jnp.int32)]).reshape(NW, K, C)
  dst = jnp.concatenate(
      [edge_index[1], jnp.full((pad,), NPAD - 1, jnp.int32)]).reshape(NW, K, C)
  z = jnp.zeros((NPAD, D), jnp.float32)
  ones = jnp.ones((C, D), jnp.float32)

  deg = _deg()(dst, z, ones)
  scale = _scale(deg)
  agg1 = _seg_sum()(x, src, dst, z)
  h1 = _dense(agg1, scale, x, W1l, W1r, b1, g1, be1)
  agg2 = _seg_sum()(h1, src, dst, z)
  h2 = _dense(agg2, scale, h1, W2l, W2r, b2, g2, be2)
  agg3 = _seg_sum()(h2, src, dst, z)
  return _final(agg3, scale, h2, W3l, W3r, b3)


# speedup vs baseline: 4.0259x; 1.2703x over previous
import jax
import jax.numpy as jnp
from jax import lax
from jax.experimental import pallas as pl
from jax.experimental.pallas import tpu as pltpu
from jax.experimental.pallas import tpu_sc as plsc

N = 10000
E = 320000
D = 128

NC = 2
NS = 16
NW = NC * NS

C = 64
K = 158
E_PAD = NW * K * C
ROWS_PER_SUB = 632
NPAD = NS * ROWS_PER_SUB

import functools


def _mesh():
  return plsc.VectorSubcoreMesh(core_axis_name="c", subcore_axis_name="s")


def _seg_sum_body(h_hbm, src_hbm, dst_hbm, z_hbm, agg_out,
                  src_v, dst_v, msg_a, msg_b, agg_sh, sem_a, sem_b):
  c = lax.axis_index("c")
  s = lax.axis_index("s")
  wid = c * NS + s
  r0 = pl.multiple_of(s * ROWS_PER_SUB, 8)

  pltpu.sync_copy(src_hbm.at[wid], src_v)
  pltpu.sync_copy(dst_hbm.at[wid], dst_v)
  pltpu.sync_copy(z_hbm.at[pl.ds(r0, ROWS_PER_SUB)],
                  agg_sh.at[pl.ds(r0, ROWS_PER_SUB)])

  plsc.subcore_barrier()

  def gather(j, buf, sem):
    pltpu.async_copy(h_hbm.at[src_v.at[j]], buf, sem)

  def wait_gather(j, buf, sem):
    pltpu.make_async_copy(h_hbm.at[src_v.at[j]], buf, sem).wait()

  def chunk(j, _):
    gather(j, msg_a, sem_a)
    wait_gather(j, msg_a, sem_a)
    pltpu.sync_copy(msg_a, agg_sh.at[dst_v.at[j]], add=True)
    return 0

  lax.fori_loop(0, K, chunk, 0)

  plsc.subcore_barrier()

  pltpu.sync_copy(agg_sh.at[pl.ds(r0, ROWS_PER_SUB)],
                  agg_out.at[c, pl.ds(r0, ROWS_PER_SUB)])


@functools.cache
def _seg_sum():
  return pl.kernel(
      _seg_sum_body,
      out_type=jax.ShapeDtypeStruct((NC, NPAD, D), jnp.float32),
      mesh=_mesh(),
      scratch_types=[
          pltpu.VMEM((K, C), jnp.int32),
          pltpu.VMEM((K, C), jnp.int32),
          pltpu.VMEM((C, D), jnp.float32),
          pltpu.VMEM((C, D), jnp.float32),
          pltpu.VMEM_SHARED((NPAD, D), jnp.float32),
          pltpu.SemaphoreType.DMA,
          pltpu.SemaphoreType.DMA,
      ],
      name="sage_seg_sum")


def _deg_body(dst_hbm, z_hbm, ones_hbm, deg_out, dst_v, ones_v, deg_sh):
  c = lax.axis_index("c")
  s = lax.axis_index("s")
  wid = c * NS + s
  r0 = pl.multiple_of(s * ROWS_PER_SUB, 8)

  pltpu.sync_copy(z_hbm.at[pl.ds(r0, ROWS_PER_SUB)],
                  deg_sh.at[pl.ds(r0, ROWS_PER_SUB)])
  pltpu.sync_copy(ones_hbm, ones_v)
  pltpu.sync_copy(dst_hbm.at[wid], dst_v)

  plsc.subcore_barrier()

  def chunk(j, _):
    pltpu.sync_copy(ones_v, deg_sh.at[dst_v.at[j]], add=True)
    return 0

  lax.fori_loop(0, K, chunk, 0)

  plsc.subcore_barrier()

  pltpu.sync_copy(deg_sh.at[pl.ds(r0, ROWS_PER_SUB)],
                  deg_out.at[c, pl.ds(r0, ROWS_PER_SUB)])


@functools.cache
def _deg():
  return pl.kernel(
      _deg_body,
      out_type=jax.ShapeDtypeStruct((NC, NPAD, D), jnp.float32),
      mesh=_mesh(),
      scratch_types=[
          pltpu.VMEM((K, C), jnp.int32),
          pltpu.VMEM((C, D), jnp.float32),
          pltpu.VMEM_SHARED((NPAD, D), jnp.float32),
      ],
      name="sage_deg")


def _dense_body(agg_ref, scale_ref, h_ref, wl_ref, wr_ref, b_ref, g_ref,
                be_ref, o_ref):
  agg = agg_ref[0, :N, :] + agg_ref[1, :N, :]
  mean = agg * scale_ref[...]
  out = (jnp.dot(mean, wl_ref[...], preferred_element_type=jnp.float32)
         + jnp.dot(h_ref[...], wr_ref[...], preferred_element_type=jnp.float32)
         + b_ref[...])
  mu = jnp.mean(out, axis=0)
  var = jnp.mean((out - mu) ** 2, axis=0)
  inv = lax.rsqrt(var + 1e-5) * g_ref[...]
  o_ref[...] = jnp.maximum((out - mu) * inv + be_ref[...], 0.0)


def _scale_body(deg_ref, scale_ref):
  deg = deg_ref[0, :N, 0:1] + deg_ref[1, :N, 0:1]
  scale_ref[...] = 1.0 / jnp.maximum(deg, 1.0)


def _final_body(agg_ref, scale_ref, h_ref, wl_ref, wr_ref, b_ref, o_ref):
  agg = agg_ref[0, :N, :] + agg_ref[1, :N, :]
  mean = agg * scale_ref[...]
  out = (jnp.dot(mean, wl_ref[...], preferred_element_type=jnp.float32)
         + jnp.dot(h_ref[...], wr_ref[...], preferred_element_type=jnp.float32)
         + b_ref[...])
  m = jnp.max(out, axis=-1, keepdims=True)
  lse = jnp.log(jnp.sum(jnp.exp(out - m), axis=-1, keepdims=True))
  o_ref[...] = out - m - lse


_dense = pl.pallas_call(
    _dense_body, out_shape=jax.ShapeDtypeStruct((N, D), jnp.float32))
_scale = pl.pallas_call(
    _scale_body, out_shape=jax.ShapeDtypeStruct((N, 1), jnp.float32))
_final = pl.pallas_call(
    _final_body, out_shape=jax.ShapeDtypeStruct((N, 64), jnp.float32))


@jax.jit
def kernel(x, edge_index, relations, W1l, W1r, b1, g1, be1, W2l, W2r, b2, g2,
           be2, W3l, W3r, b3):
  del relations
  pad = E_PAD - E
  src = jnp.concatenate(
      [edge_index[0], jnp.zeros((pad,), jnp.int32)]).reshape(NW, K, C)
  dst = jnp.concatenate(
      [edge_index[1], jnp.full((pad,), NPAD - 1, jnp.int32)]).reshape(NW, K, C)
  z = jnp.zeros((NPAD, D), jnp.float32)
  ones = jnp.ones((C, D), jnp.float32)

  deg = _deg()(dst, z, ones)
  scale = _scale(deg)
  agg1 = _seg_sum()(x, src, dst, z)
  h1 = _dense(agg1, scale, x, W1l, W1r, b1, g1, be1)
  agg2 = _seg_sum()(h1, src, dst, z)
  h2 = _dense(agg2, scale, h1, W2l, W2r, b2, g2, be2)
  agg3 = _seg_sum()(h2, src, dst, z)
  return _final(agg3, scale, h2, W3l, W3r, b3)


# speedup vs baseline: 5.1891x; 1.2889x over previous
import jax
import jax.numpy as jnp
from jax import lax
from jax.experimental import pallas as pl
from jax.experimental.pallas import tpu as pltpu
from jax.experimental.pallas import tpu_sc as plsc

N = 10000
E = 320000
D = 128

NC = 2
NS = 16
NW = NC * NS

C = 96
K = 105
E_PAD = NW * K * C
ROWS_PER_SUB = 632
NPAD = NS * ROWS_PER_SUB

import functools


def _mesh():
  return plsc.VectorSubcoreMesh(core_axis_name="c", subcore_axis_name="s")


def _seg_sum_body(h_hbm, src_hbm, dst_hbm, z_hbm, agg_out,
                  src_v, dst_v, msg_a, msg_b, agg_sh, sem_a, sem_b):
  c = lax.axis_index("c")
  s = lax.axis_index("s")
  wid = c * NS + s
  r0 = pl.multiple_of(s * ROWS_PER_SUB, 8)

  pltpu.sync_copy(src_hbm.at[wid], src_v)
  pltpu.sync_copy(dst_hbm.at[wid], dst_v)
  pltpu.sync_copy(z_hbm.at[pl.ds(r0, ROWS_PER_SUB)],
                  agg_sh.at[pl.ds(r0, ROWS_PER_SUB)])

  plsc.subcore_barrier()

  def gather(j, buf, sem):
    pltpu.async_copy(h_hbm.at[src_v.at[j]], buf, sem)

  def wait_gather(j, buf, sem):
    pltpu.make_async_copy(h_hbm.at[src_v.at[j]], buf, sem).wait()

  def chunk(j, _):
    gather(j, msg_a, sem_a)
    wait_gather(j, msg_a, sem_a)
    pltpu.sync_copy(msg_a, agg_sh.at[dst_v.at[j]], add=True)
    return 0

  lax.fori_loop(0, K, chunk, 0)

  plsc.subcore_barrier()

  pltpu.sync_copy(agg_sh.at[pl.ds(r0, ROWS_PER_SUB)],
                  agg_out.at[c, pl.ds(r0, ROWS_PER_SUB)])


@functools.cache
def _seg_sum():
  return pl.kernel(
      _seg_sum_body,
      out_type=jax.ShapeDtypeStruct((NC, NPAD, D), jnp.float32),
      mesh=_mesh(),
      scratch_types=[
          pltpu.VMEM((K, C), jnp.int32),
          pltpu.VMEM((K, C), jnp.int32),
          pltpu.VMEM((C, D), jnp.float32),
          pltpu.VMEM((C, D), jnp.float32),
          pltpu.VMEM_SHARED((NPAD, D), jnp.float32),
          pltpu.SemaphoreType.DMA,
          pltpu.SemaphoreType.DMA,
      ],
      name="sage_seg_sum")


def _deg_body(dst_hbm, z_hbm, ones_hbm, deg_out, dst_v, ones_v, deg_sh):
  c = lax.axis_index("c")
  s = lax.axis_index("s")
  wid = c * NS + s
  r0 = pl.multiple_of(s * ROWS_PER_SUB, 8)

  pltpu.sync_copy(z_hbm.at[pl.ds(r0, ROWS_PER_SUB)],
                  deg_sh.at[pl.ds(r0, ROWS_PER_SUB)])
  pltpu.sync_copy(ones_hbm, ones_v)
  pltpu.sync_copy(dst_hbm.at[wid], dst_v)

  plsc.subcore_barrier()

  def chunk(j, _):
    pltpu.sync_copy(ones_v, deg_sh.at[dst_v.at[j]], add=True)
    return 0

  lax.fori_loop(0, K, chunk, 0)

  plsc.subcore_barrier()

  pltpu.sync_copy(deg_sh.at[pl.ds(r0, ROWS_PER_SUB)],
                  deg_out.at[c, pl.ds(r0, ROWS_PER_SUB)])


@functools.cache
def _deg():
  return pl.kernel(
      _deg_body,
      out_type=jax.ShapeDtypeStruct((NC, NPAD, D), jnp.float32),
      mesh=_mesh(),
      scratch_types=[
          pltpu.VMEM((K, C), jnp.int32),
          pltpu.VMEM((C, D), jnp.float32),
          pltpu.VMEM_SHARED((NPAD, D), jnp.float32),
      ],
      name="sage_deg")


def _dense_body(agg_ref, scale_ref, h_ref, wl_ref, wr_ref, b_ref, g_ref,
                be_ref, o_ref):
  agg = agg_ref[0, :N, :] + agg_ref[1, :N, :]
  mean = agg * scale_ref[...]
  out = (jnp.dot(mean, wl_ref[...], preferred_element_type=jnp.float32)
         + jnp.dot(h_ref[...], wr_ref[...], preferred_element_type=jnp.float32)
         + b_ref[...])
  mu = jnp.mean(out, axis=0)
  var = jnp.mean((out - mu) ** 2, axis=0)
  inv = lax.rsqrt(var + 1e-5) * g_ref[...]
  o_ref[...] = jnp.maximum((out - mu) * inv + be_ref[...], 0.0)


def _scale_body(deg_ref, scale_ref):
  deg = deg_ref[0, :N, 0:1] + deg_ref[1, :N, 0:1]
  scale_ref[...] = 1.0 / jnp.maximum(deg, 1.0)


def _final_body(agg_ref, scale_ref, h_ref, wl_ref, wr_ref, b_ref, o_ref):
  agg = agg_ref[0, :N, :] + agg_ref[1, :N, :]
  mean = agg * scale_ref[...]
  out = (jnp.dot(mean, wl_ref[...], preferred_element_type=jnp.float32)
         + jnp.dot(h_ref[...], wr_ref[...], preferred_element_type=jnp.float32)
         + b_ref[...])
  m = jnp.max(out, axis=-1, keepdims=True)
  lse = jnp.log(jnp.sum(jnp.exp(out - m), axis=-1, keepdims=True))
  o_ref[...] = out - m - lse


_dense = pl.pallas_call(
    _dense_body, out_shape=jax.ShapeDtypeStruct((N, D), jnp.float32))
_scale = pl.pallas_call(
    _scale_body, out_shape=jax.ShapeDtypeStruct((N, 1), jnp.float32))
_final = pl.pallas_call(
    _final_body, out_shape=jax.ShapeDtypeStruct((N, 64), jnp.float32))


@jax.jit
def kernel(x, edge_index, relations, W1l, W1r, b1, g1, be1, W2l, W2r, b2, g2,
           be2, W3l, W3r, b3):
  del relations
  pad = E_PAD - E
  src = jnp.concatenate(
      [edge_index[0], jnp.zeros((pad,), jnp.int32)]).reshape(NW, K, C)
  dst = jnp.concatenate(
      [edge_index[1], jnp.full((pad,), NPAD - 1, jnp.int32)]).reshape(NW, K, C)
  z = jnp.zeros((NPAD, D), jnp.float32)
  ones = jnp.ones((C, D), jnp.float32)

  deg = _deg()(dst, z, ones)
  scale = _scale(deg)
  agg1 = _seg_sum()(x, src, dst, z)
  h1 = _dense(agg1, scale, x, W1l, W1r, b1, g1, be1)
  agg2 = _seg_sum()(h1, src, dst, z)
  h2 = _dense(agg2, scale, h1, W2l, W2r, b2, g2, be2)
  agg3 = _seg_sum()(h2, src, dst, z)
  return _final(agg3, scale, h2, W3l, W3r, b3)


# speedup vs baseline: 5.2212x; 1.0062x over previous
import jax
import jax.numpy as jnp
from jax import lax
from jax.experimental import pallas as pl
from jax.experimental.pallas import tpu as pltpu
from jax.experimental.pallas import tpu_sc as plsc

N = 10000
E = 320000
D = 128

NC = 2
NS = 16
NW = NC * NS

C = 112
K = 90
E_PAD = NW * K * C
ROWS_PER_SUB = 632
NPAD = NS * ROWS_PER_SUB

import functools


def _mesh():
  return plsc.VectorSubcoreMesh(core_axis_name="c", subcore_axis_name="s")


def _seg_sum_body(h_hbm, src_hbm, dst_hbm, z_hbm, agg_out,
                  src_v, dst_v, msg_a, msg_b, agg_sh, sem_a, sem_b):
  c = lax.axis_index("c")
  s = lax.axis_index("s")
  wid = c * NS + s
  r0 = pl.multiple_of(s * ROWS_PER_SUB, 8)

  pltpu.sync_copy(src_hbm.at[wid], src_v)
  pltpu.sync_copy(dst_hbm.at[wid], dst_v)
  pltpu.sync_copy(z_hbm.at[pl.ds(r0, ROWS_PER_SUB)],
                  agg_sh.at[pl.ds(r0, ROWS_PER_SUB)])

  plsc.subcore_barrier()

  def gather(j, buf, sem):
    pltpu.async_copy(h_hbm.at[src_v.at[j]], buf, sem)

  def wait_gather(j, buf, sem):
    pltpu.make_async_copy(h_hbm.at[src_v.at[j]], buf, sem).wait()

  def chunk(j, _):
    gather(j, msg_a, sem_a)
    wait_gather(j, msg_a, sem_a)
    pltpu.sync_copy(msg_a, agg_sh.at[dst_v.at[j]], add=True)
    return 0

  lax.fori_loop(0, K, chunk, 0)

  plsc.subcore_barrier()

  pltpu.sync_copy(agg_sh.at[pl.ds(r0, ROWS_PER_SUB)],
                  agg_out.at[c, pl.ds(r0, ROWS_PER_SUB)])


@functools.cache
def _seg_sum():
  return pl.kernel(
      _seg_sum_body,
      out_type=jax.ShapeDtypeStruct((NC, NPAD, D), jnp.float32),
      mesh=_mesh(),
      scratch_types=[
          pltpu.VMEM((K, C), jnp.int32),
          pltpu.VMEM((K, C), jnp.int32),
          pltpu.VMEM((C, D), jnp.float32),
          pltpu.VMEM((C, D), jnp.float32),
          pltpu.VMEM_SHARED((NPAD, D), jnp.float32),
          pltpu.SemaphoreType.DMA,
          pltpu.SemaphoreType.DMA,
      ],
      name="sage_seg_sum")


def _deg_body(dst_hbm, z_hbm, ones_hbm, deg_out, dst_v, ones_v, deg_sh):
  c = lax.axis_index("c")
  s = lax.axis_index("s")
  wid = c * NS + s
  r0 = pl.multiple_of(s * ROWS_PER_SUB, 8)

  pltpu.sync_copy(z_hbm.at[pl.ds(r0, ROWS_PER_SUB)],
                  deg_sh.at[pl.ds(r0, ROWS_PER_SUB)])
  pltpu.sync_copy(ones_hbm, ones_v)
  pltpu.sync_copy(dst_hbm.at[wid], dst_v)

  plsc.subcore_barrier()

  def chunk(j, _):
    pltpu.sync_copy(ones_v, deg_sh.at[dst_v.at[j]], add=True)
    return 0

  lax.fori_loop(0, K, chunk, 0)

  plsc.subcore_barrier()

  pltpu.sync_copy(deg_sh.at[pl.ds(r0, ROWS_PER_SUB)],
                  deg_out.at[c, pl.ds(r0, ROWS_PER_SUB)])


@functools.cache
def _deg():
  return pl.kernel(
      _deg_body,
      out_type=jax.ShapeDtypeStruct((NC, NPAD, D), jnp.float32),
      mesh=_mesh(),
      scratch_types=[
          pltpu.VMEM((K, C), jnp.int32),
          pltpu.VMEM((C, D), jnp.float32),
          pltpu.VMEM_SHARED((NPAD, D), jnp.float32),
      ],
      name="sage_deg")


def _dense_body(agg_ref, scale_ref, h_ref, wl_ref, wr_ref, b_ref, g_ref,
                be_ref, o_ref):
  agg = agg_ref[0, :N, :] + agg_ref[1, :N, :]
  mean = agg * scale_ref[...]
  out = (jnp.dot(mean, wl_ref[...], preferred_element_type=jnp.float32)
         + jnp.dot(h_ref[...], wr_ref[...], preferred_element_type=jnp.float32)
         + b_ref[...])
  mu = jnp.mean(out, axis=0)
  var = jnp.mean((out - mu) ** 2, axis=0)
  inv = lax.rsqrt(var + 1e-5) * g_ref[...]
  o_ref[...] = jnp.maximum((out - mu) * inv + be_ref[...], 0.0)


def _scale_body(deg_ref, scale_ref):
  deg = deg_ref[0, :N, 0:1] + deg_ref[1, :N, 0:1]
  scale_ref[...] = 1.0 / jnp.maximum(deg, 1.0)


def _final_body(agg_ref, scale_ref, h_ref, wl_ref, wr_ref, b_ref, o_ref):
  agg = agg_ref[0, :N, :] + agg_ref[1, :N, :]
  mean = agg * scale_ref[...]
  out = (jnp.dot(mean, wl_ref[...], preferred_element_type=jnp.float32)
         + jnp.dot(h_ref[...], wr_ref[...], preferred_element_type=jnp.float32)
         + b_ref[...])
  m = jnp.max(out, axis=-1, keepdims=True)
  lse = jnp.log(jnp.sum(jnp.exp(out - m), axis=-1, keepdims=True))
  o_ref[...] = out - m - lse


_dense = pl.pallas_call(
    _dense_body, out_shape=jax.ShapeDtypeStruct((N, D), jnp.float32))
_scale = pl.pallas_call(
    _scale_body, out_shape=jax.ShapeDtypeStruct((N, 1), jnp.float32))
_final = pl.pallas_call(
    _final_body, out_shape=jax.ShapeDtypeStruct((N, 64), jnp.float32))


@jax.jit
def kernel(x, edge_index, relations, W1l, W1r, b1, g1, be1, W2l, W2r, b2, g2,
           be2, W3l, W3r, b3):
  del relations
  pad = E_PAD - E
  src = jnp.concatenate(
      [edge_index[0], jnp.zeros((pad,), jnp.int32)]).reshape(NW, K, C)
  dst = jnp.concatenate(
      [edge_index[1], jnp.full((pad,), NPAD - 1, jnp.int32)]).reshape(NW, K, C)
  z = jnp.zeros((NPAD, D), jnp.float32)
  ones = jnp.ones((C, D), jnp.float32)

  deg = _deg()(dst, z, ones)
  scale = _scale(deg)
  agg1 = _seg_sum()(x, src, dst, z)
  h1 = _dense(agg1, scale, x, W1l, W1r, b1, g1, be1)
  agg2 = _seg_sum()(h1, src, dst, z)
  h2 = _dense(agg2, scale, h1, W2l, W2r, b2, g2, be2)
  agg3 = _seg_sum()(h2, src, dst, z)
  return _final(agg3, scale, h2, W3l, W3r, b3)


# speedup vs baseline: 5.4339x; 1.0407x over previous
import jax
import jax.numpy as jnp
from jax import lax
from jax.experimental import pallas as pl
from jax.experimental.pallas import tpu as pltpu
from jax.experimental.pallas import tpu_sc as plsc

N = 10000
E = 320000
D = 128

NC = 2
NS = 16
NW = NC * NS

C = 120
K = 84
E_PAD = NW * K * C
ROWS_PER_SUB = 632
NPAD = NS * ROWS_PER_SUB

import functools


def _mesh():
  return plsc.VectorSubcoreMesh(core_axis_name="c", subcore_axis_name="s")


def _seg_sum_body(h_hbm, src_hbm, dst_hbm, z_hbm, agg_out,
                  src_v, dst_v, msg_a, msg_b, agg_sh, sem_a, sem_b):
  c = lax.axis_index("c")
  s = lax.axis_index("s")
  wid = c * NS + s
  r0 = pl.multiple_of(s * ROWS_PER_SUB, 8)

  pltpu.sync_copy(src_hbm.at[wid], src_v)
  pltpu.sync_copy(dst_hbm.at[wid], dst_v)
  pltpu.sync_copy(z_hbm.at[pl.ds(r0, ROWS_PER_SUB)],
                  agg_sh.at[pl.ds(r0, ROWS_PER_SUB)])

  plsc.subcore_barrier()

  def gather(j, buf, sem):
    pltpu.async_copy(h_hbm.at[src_v.at[j]], buf, sem)

  def wait_gather(j, buf, sem):
    pltpu.make_async_copy(h_hbm.at[src_v.at[j]], buf, sem).wait()

  def chunk(j, _):
    gather(j, msg_a, sem_a)
    wait_gather(j, msg_a, sem_a)
    pltpu.sync_copy(msg_a, agg_sh.at[dst_v.at[j]], add=True)
    return 0

  lax.fori_loop(0, K, chunk, 0)

  plsc.subcore_barrier()

  pltpu.sync_copy(agg_sh.at[pl.ds(r0, ROWS_PER_SUB)],
                  agg_out.at[c, pl.ds(r0, ROWS_PER_SUB)])


@functools.cache
def _seg_sum():
  return pl.kernel(
      _seg_sum_body,
      out_type=jax.ShapeDtypeStruct((NC, NPAD, D), jnp.float32),
      mesh=_mesh(),
      scratch_types=[
          pltpu.VMEM((K, C), jnp.int32),
          pltpu.VMEM((K, C), jnp.int32),
          pltpu.VMEM((C, D), jnp.float32),
          pltpu.VMEM((C, D), jnp.float32),
          pltpu.VMEM_SHARED((NPAD, D), jnp.float32),
          pltpu.SemaphoreType.DMA,
          pltpu.SemaphoreType.DMA,
      ],
      name="sage_seg_sum")


def _deg_body(dst_hbm, z_hbm, ones_hbm, deg_out, dst_v, ones_v, deg_sh):
  c = lax.axis_index("c")
  s = lax.axis_index("s")
  wid = c * NS + s
  r0 = pl.multiple_of(s * ROWS_PER_SUB, 8)

  pltpu.sync_copy(z_hbm.at[pl.ds(r0, ROWS_PER_SUB)],
                  deg_sh.at[pl.ds(r0, ROWS_PER_SUB)])
  pltpu.sync_copy(ones_hbm, ones_v)
  pltpu.sync_copy(dst_hbm.at[wid], dst_v)

  plsc.subcore_barrier()

  def chunk(j, _):
    pltpu.sync_copy(ones_v, deg_sh.at[dst_v.at[j]], add=True)
    return 0

  lax.fori_loop(0, K, chunk, 0)

  plsc.subcore_barrier()

  pltpu.sync_copy(deg_sh.at[pl.ds(r0, ROWS_PER_SUB)],
                  deg_out.at[c, pl.ds(r0, ROWS_PER_SUB)])


@functools.cache
def _deg():
  return pl.kernel(
      _deg_body,
      out_type=jax.ShapeDtypeStruct((NC, NPAD, D), jnp.float32),
      mesh=_mesh(),
      scratch_types=[
          pltpu.VMEM((K, C), jnp.int32),
          pltpu.VMEM((C, D), jnp.float32),
          pltpu.VMEM_SHARED((NPAD, D), jnp.float32),
      ],
      name="sage_deg")


def _dense_body(agg_ref, scale_ref, h_ref, wl_ref, wr_ref, b_ref, g_ref,
                be_ref, o_ref):
  agg = agg_ref[0, :N, :] + agg_ref[1, :N, :]
  mean = agg * scale_ref[...]
  out = (jnp.dot(mean, wl_ref[...], preferred_element_type=jnp.float32)
         + jnp.dot(h_ref[...], wr_ref[...], preferred_element_type=jnp.float32)
         + b_ref[...])
  mu = jnp.mean(out, axis=0)
  var = jnp.mean((out - mu) ** 2, axis=0)
  inv = lax.rsqrt(var + 1e-5) * g_ref[...]
  o_ref[...] = jnp.maximum((out - mu) * inv + be_ref[...], 0.0)


def _scale_body(deg_ref, scale_ref):
  deg = deg_ref[0, :N, 0:1] + deg_ref[1, :N, 0:1]
  scale_ref[...] = 1.0 / jnp.maximum(deg, 1.0)


def _final_body(agg_ref, scale_ref, h_ref, wl_ref, wr_ref, b_ref, o_ref):
  agg = agg_ref[0, :N, :] + agg_ref[1, :N, :]
  mean = agg * scale_ref[...]
  out = (jnp.dot(mean, wl_ref[...], preferred_element_type=jnp.float32)
         + jnp.dot(h_ref[...], wr_ref[...], preferred_element_type=jnp.float32)
         + b_ref[...])
  m = jnp.max(out, axis=-1, keepdims=True)
  lse = jnp.log(jnp.sum(jnp.exp(out - m), axis=-1, keepdims=True))
  o_ref[...] = out - m - lse


_dense = pl.pallas_call(
    _dense_body, out_shape=jax.ShapeDtypeStruct((N, D), jnp.float32))
_scale = pl.pallas_call(
    _scale_body, out_shape=jax.ShapeDtypeStruct((N, 1), jnp.float32))
_final = pl.pallas_call(
    _final_body, out_shape=jax.ShapeDtypeStruct((N, 64), jnp.float32))


@jax.jit
def kernel(x, edge_index, relations, W1l, W1r, b1, g1, be1, W2l, W2r, b2, g2,
           be2, W3l, W3r, b3):
  del relations
  pad = E_PAD - E
  src = jnp.concatenate(
      [edge_index[0], jnp.zeros((pad,), jnp.int32)]).reshape(NW, K, C)
  dst = jnp.concatenate(
      [edge_index[1], jnp.full((pad,), NPAD - 1, jnp.int32)]).reshape(NW, K, C)
  z = jnp.zeros((NPAD, D), jnp.float32)
  ones = jnp.ones((C, D), jnp.float32)

  deg = _deg()(dst, z, ones)
  scale = _scale(deg)
  agg1 = _seg_sum()(x, src, dst, z)
  h1 = _dense(agg1, scale, x, W1l, W1r, b1, g1, be1)
  agg2 = _seg_sum()(h1, src, dst, z)
  h2 = _dense(agg2, scale, h1, W2l, W2r, b2, g2, be2)
  agg3 = _seg_sum()(h2, src, dst, z)
  return _final(agg3, scale, h2, W3l, W3r, b3)


# speedup vs baseline: 5.4342x; 1.0001x over previous
import jax
import jax.numpy as jnp
from jax import lax
from jax.experimental import pallas as pl
from jax.experimental.pallas import tpu as pltpu
from jax.experimental.pallas import tpu_sc as plsc

N = 10000
E = 320000
D = 128

NC = 2
NS = 16
NW = NC * NS

C = 120
K = 84
E_PAD = NW * K * C
ROWS_PER_SUB = 632
NPAD = NS * ROWS_PER_SUB

import functools


def _mesh():
  return plsc.VectorSubcoreMesh(core_axis_name="c", subcore_axis_name="s")


def _seg_sum_body(h_hbm, src_hbm, dst_hbm, z_hbm, agg_out,
                  src_v, dst_v, msg_a, msg_b, agg_sh, sem_a, sem_b):
  c = lax.axis_index("c")
  s = lax.axis_index("s")
  wid = c * NS + s
  r0 = pl.multiple_of(s * ROWS_PER_SUB, 8)

  pltpu.sync_copy(src_hbm.at[wid], src_v)
  pltpu.sync_copy(dst_hbm.at[wid], dst_v)
  pltpu.sync_copy(z_hbm.at[pl.ds(r0, ROWS_PER_SUB)],
                  agg_sh.at[pl.ds(r0, ROWS_PER_SUB)])

  plsc.subcore_barrier()

  def gather(j, buf, sem):
    pltpu.async_copy(h_hbm.at[src_v.at[j]], buf, sem)

  def wait_gather(j, buf, sem):
    pltpu.make_async_copy(h_hbm.at[src_v.at[j]], buf, sem).wait()

  def chunk(j, _):
    gather(j, msg_a, sem_a)
    wait_gather(j, msg_a, sem_a)
    pltpu.sync_copy(msg_a, agg_sh.at[dst_v.at[j]], add=True)
    return 0

  lax.fori_loop(0, K, chunk, 0)

  plsc.subcore_barrier()

  pltpu.sync_copy(agg_sh.at[pl.ds(r0, ROWS_PER_SUB)],
                  agg_out.at[c, pl.ds(r0, ROWS_PER_SUB)])


@functools.cache
def _seg_sum():
  return pl.kernel(
      _seg_sum_body,
      out_type=jax.ShapeDtypeStruct((NC, NPAD, D), jnp.float32),
      mesh=_mesh(),
      scratch_types=[
          pltpu.VMEM((K, C), jnp.int32),
          pltpu.VMEM((K, C), jnp.int32),
          pltpu.VMEM((C, D), jnp.float32),
          pltpu.VMEM((C, D), jnp.float32),
          pltpu.VMEM_SHARED((NPAD, D), jnp.float32),
          pltpu.SemaphoreType.DMA,
          pltpu.SemaphoreType.DMA,
      ],
      name="sage_seg_sum")


def _deg_body(dst_hbm, z_hbm, ones_hbm, deg_out, dst_v, ones_v, deg_sh):
  c = lax.axis_index("c")
  s = lax.axis_index("s")
  wid = c * NS + s
  r0 = pl.multiple_of(s * ROWS_PER_SUB, 8)

  pltpu.sync_copy(z_hbm.at[pl.ds(r0, ROWS_PER_SUB)],
                  deg_sh.at[pl.ds(r0, ROWS_PER_SUB)])
  pltpu.sync_copy(ones_hbm, ones_v)
  pltpu.sync_copy(dst_hbm.at[wid], dst_v)

  plsc.subcore_barrier()

  def chunk(j, _):
    pltpu.sync_copy(ones_v, deg_sh.at[dst_v.at[j]], add=True)
    return 0

  lax.fori_loop(0, K, chunk, 0)

  plsc.subcore_barrier()

  pltpu.sync_copy(deg_sh.at[pl.ds(r0, ROWS_PER_SUB)],
                  deg_out.at[c, pl.ds(r0, ROWS_PER_SUB)])


@functools.cache
def _deg():
  return pl.kernel(
      _deg_body,
      out_type=jax.ShapeDtypeStruct((NC, NPAD, D), jnp.float32),
      mesh=_mesh(),
      scratch_types=[
          pltpu.VMEM((K, C), jnp.int32),
          pltpu.VMEM((C, D), jnp.float32),
          pltpu.VMEM_SHARED((NPAD, D), jnp.float32),
      ],
      name="sage_deg")


def _dense_body(agg_ref, scale_ref, h_ref, wl_ref, wr_ref, b_ref, g_ref,
                be_ref, o_ref):
  agg = agg_ref[0, :N, :] + agg_ref[1, :N, :]
  mean = agg * scale_ref[...]
  out = (jnp.dot(mean, wl_ref[...], preferred_element_type=jnp.float32)
         + jnp.dot(h_ref[...], wr_ref[...], preferred_element_type=jnp.float32)
         + b_ref[...])
  mu = jnp.mean(out, axis=0)
  var = jnp.mean((out - mu) ** 2, axis=0)
  inv = lax.rsqrt(var + 1e-5) * g_ref[...]
  o_ref[...] = jnp.maximum((out - mu) * inv + be_ref[...], 0.0)


def _scale_body(deg_ref, scale_ref):
  deg = deg_ref[0, :N, 0:1] + deg_ref[1, :N, 0:1]
  scale_ref[...] = 1.0 / jnp.maximum(deg, 1.0)


def _final_body(agg_ref, scale_ref, h_ref, wl_ref, wr_ref, b_ref, o_ref):
  agg = agg_ref[0, :N, :] + agg_ref[1, :N, :]
  mean = agg * scale_ref[...]
  out = (jnp.dot(mean, wl_ref[...], preferred_element_type=jnp.float32)
         + jnp.dot(h_ref[...], wr_ref[...], preferred_element_type=jnp.float32)
         + b_ref[...])
  m = jnp.max(out, axis=-1, keepdims=True)
  lse = jnp.log(jnp.sum(jnp.exp(out - m), axis=-1, keepdims=True))
  o_ref[...] = out - m - lse


_dense = pl.pallas_call(
    _dense_body, out_shape=jax.ShapeDtypeStruct((N, D), jnp.float32))
_scale = pl.pallas_call(
    _scale_body, out_shape=jax.ShapeDtypeStruct((N, 1), jnp.float32))
_final = pl.pallas_call(
    _final_body, out_shape=jax.ShapeDtypeStruct((N, 64), jnp.float32))


@jax.jit
def kernel(x, edge_index, relations, W1l, W1r, b1, g1, be1, W2l, W2r, b2, g2,
           be2, W3l, W3r, b3):
  del relations
  pad = E_PAD - E
  src = jnp.concatenate(
      [edge_index[0], jnp.zeros((pad,), jnp.int32)]).reshape(NW, K, C)
  pad_dst = N + (jnp.arange(pad, dtype=jnp.int32) % (NPAD - N))
  dst = jnp.concatenate([edge_index[1], pad_dst]).reshape(NW, K, C)
  z = jnp.zeros((NPAD, D), jnp.float32)
  ones = jnp.ones((C, D), jnp.float32)

  deg = _deg()(dst, z, ones)
  scale = _scale(deg)
  agg1 = _seg_sum()(x, src, dst, z)
  h1 = _dense(agg1, scale, x, W1l, W1r, b1, g1, be1)
  agg2 = _seg_sum()(h1, src, dst, z)
  h2 = _dense(agg2, scale, h1, W2l, W2r, b2, g2, be2)
  agg3 = _seg_sum()(h2, src, dst, z)
  return _final(agg3, scale, h2, W3l, W3r, b3)
